# baseline (speedup 1.0000x reference)
import jax
import jax.numpy as jnp
from jax import lax
from jax.experimental import pallas as pl
from jax.experimental.pallas import tpu as pltpu
from jax.experimental.pallas import tpu_sc as plsc

N_NODES = 10000
D = 128
E = 320000

NC = 2
NS = 16
NW = NC * NS
K = 128
CH = 80
EW = K * CH
EP = EW * NW
NP = 10240
RPT = NP // NS
BM = 1024
TOT_CH = EP // K
GRP = 16
N_C0 = 160
N_C1 = 0

_SC_MESH = plsc.VectorSubcoreMesh(
    core_axis_name="c", subcore_axis_name="s", num_cores=NC, num_subcores=NS
)


def _deg_body(dstr, out, degw, dstv, onesb):
    c = lax.axis_index("c")
    s = lax.axis_index("s")
    wid = c * NS + s
    pltpu.sync_copy(dstr.at[pl.ds(wid * CH, CH)], dstv)

    def _fill0(i, carry):
        for kk in range(D // 16):
            onesb[i, pl.ds(kk * 16, 16)] = jnp.zeros((16,), jnp.float32)
        return carry

    lax.fori_loop(0, K, _fill0, 0)
    base = s * RPT
    for z in range(RPT // K):
        pltpu.sync_copy(onesb, degw.at[pl.ds(base + z * K, K)])

    def _fill1(i, carry):
        for kk in range(D // 16):
            onesb[i, pl.ds(kk * 16, 16)] = jnp.ones((16,), jnp.float32)
        return carry

    lax.fori_loop(0, K, _fill1, 0)
    plsc.subcore_barrier()

    def _chunk(t, carry):
        pltpu.sync_copy(onesb, degw.at[dstv.at[t]], add=True)
        return carry

    lax.fori_loop(0, CH, _chunk, 0)
    plsc.subcore_barrier()
    for z in range(RPT // K):
        pltpu.sync_copy(degw.at[pl.ds(base + z * K, K)], onesb)
        pltpu.sync_copy(onesb, out.at[c, pl.ds(base + z * K, K)])


_deg_call = pl.kernel(
    _deg_body,
    out_type=jax.ShapeDtypeStruct((NC, NP, D), jnp.float32),
    mesh=_SC_MESH,
    scratch_types=[
        pltpu.VMEM_SHARED((NP, D), jnp.float32),
        pltpu.VMEM((CH, K), jnp.int32),
        pltpu.VMEM((K, D), jnp.float32),
    ],
)


def _edge_body(hs, srcr, dstr, out, acc, srcv, dstv, buf0, buf1, sem0, sem1, sem2, sem3):
    c = lax.axis_index("c")
    s = lax.axis_index("s")

    def _zrow(i, carry):
        for kk in range(D // 16):
            buf0[i, pl.ds(kk * 16, 16)] = jnp.zeros((16,), jnp.float32)
        return carry

    lax.fori_loop(0, K, _zrow, 0)
    base = s * RPT
    for z in range(RPT // K):
        pltpu.sync_copy(buf0, acc.at[pl.ds(base + z * K, K)])
    plsc.subcore_barrier()

    def _chunk(t, carry):
        j0 = 2 * t
        j1 = 2 * t + 1
        pltpu.make_async_copy(hs.at[srcv.at[j0]], buf0, sem0).wait()
        pltpu.async_copy(buf0, acc.at[dstv.at[j0]], sem2, add=True)
        pltpu.make_async_copy(hs.at[srcv.at[j1]], buf1, sem1).wait()
        pltpu.async_copy(buf1, acc.at[dstv.at[j1]], sem3, add=True)
        pltpu.make_async_copy(buf0, acc.at[dstv.at[j0]], sem2).wait()
        pltpu.async_copy(hs.at[srcv.at[j0 + 2]], buf0, sem0)
        pltpu.make_async_copy(buf1, acc.at[dstv.at[j1]], sem3).wait()
        pltpu.async_copy(hs.at[srcv.at[j1 + 2]], buf1, sem1)
        return carry

    def _run(base, n_chunks):
        for g in range(n_chunks // GRP):
            gb = base + g * GRP
            pltpu.sync_copy(srcr.at[pl.ds(gb, GRP)], srcv)
            pltpu.sync_copy(dstr.at[pl.ds(gb, GRP)], dstv)
            pltpu.async_copy(hs.at[srcv.at[0]], buf0, sem0)
            pltpu.async_copy(hs.at[srcv.at[1]], buf1, sem1)
            lax.fori_loop(0, GRP // 2 - 1, _chunk, 0)
            pltpu.make_async_copy(hs.at[srcv.at[GRP - 2]], buf0, sem0).wait()
            pltpu.sync_copy(buf0, acc.at[dstv.at[GRP - 2]], add=True)
            pltpu.make_async_copy(hs.at[srcv.at[GRP - 1]], buf1, sem1).wait()
            pltpu.sync_copy(buf1, acc.at[dstv.at[GRP - 1]], add=True)

    @pl.when(c == 0)
    def _core0():
        _run(s * N_C0, N_C0)

    @pl.when(c == 1)
    def _core1():
        _run(NS * N_C0 + s * N_C1, N_C1)

    plsc.subcore_barrier()
    for z in range(RPT // K):
        pltpu.sync_copy(acc.at[pl.ds(base + z * K, K)], buf0)
        pltpu.sync_copy(buf0, out.at[c, pl.ds(base + z * K, K)])


_edge_call = pl.kernel(
    _edge_body,
    out_type=jax.ShapeDtypeStruct((NC, NP, D), jnp.float32),
    mesh=_SC_MESH,
    scratch_types=[
        pltpu.VMEM_SHARED((NP, D), jnp.float32),
        pltpu.VMEM((GRP, K), jnp.int32),
        pltpu.VMEM((GRP, K), jnp.int32),
        pltpu.VMEM((K, D), jnp.float32),
        pltpu.VMEM((K, D), jnp.float32),
        pltpu.SemaphoreType.DMA,
        pltpu.SemaphoreType.DMA,
        pltpu.SemaphoreType.DMA,
        pltpu.SemaphoreType.DMA,
    ],
)


def _k1_body(x_ref, w_ref, deg_ref, hs_ref, dinv_ref):
    d = deg_ref[0, :, 0:1] + deg_ref[1, :, 0:1] + 1.0
    dv = lax.rsqrt(d)
    dinv_ref[...] = dv
    h = jnp.dot(x_ref[...], w_ref[...], preferred_element_type=jnp.float32)
    hs_ref[...] = h * dv


_k1 = pl.pallas_call(
    _k1_body,
    grid=(NP // BM,),
    in_specs=[
        pl.BlockSpec((BM, D), lambda i: (i, 0)),
        pl.BlockSpec((D, D), lambda i: (0, 0)),
        pl.BlockSpec((NC, BM, D), lambda i: (0, i, 0)),
    ],
    out_specs=[
        pl.BlockSpec((BM, D), lambda i: (i, 0)),
        pl.BlockSpec((BM, 1), lambda i: (i, 0)),
    ],
    out_shape=[
        jax.ShapeDtypeStruct((NP, D), jnp.float32),
        jax.ShapeDtypeStruct((NP, 1), jnp.float32),
    ],
)


def _k2_body(hs_ref, p_ref, dinv_ref, b_ref, w_ref, o_ref):
    dv = dinv_ref[...]
    a = (hs_ref[...] + p_ref[0] + p_ref[1]) * dv + b_ref[...]
    t = jnp.maximum(a, 0.0)
    o_ref[...] = jnp.dot(t, w_ref[...], preferred_element_type=jnp.float32) * dv


_k2 = pl.pallas_call(
    _k2_body,
    grid=(NP // BM,),
    in_specs=[
        pl.BlockSpec((BM, D), lambda i: (i, 0)),
        pl.BlockSpec((NC, BM, D), lambda i: (0, i, 0)),
        pl.BlockSpec((BM, 1), lambda i: (i, 0)),
        pl.BlockSpec((1, D), lambda i: (0, 0)),
        pl.BlockSpec((D, D), lambda i: (0, 0)),
    ],
    out_specs=pl.BlockSpec((BM, D), lambda i: (i, 0)),
    out_shape=jax.ShapeDtypeStruct((NP, D), jnp.float32),
)


def _k3_body(hs_ref, p_ref, dinv_ref, b_ref, w_ref, bl_ref, o_ref):
    a = (hs_ref[...] + p_ref[0] + p_ref[1]) * dinv_ref[...] + b_ref[...]
    t = jnp.maximum(a, 0.0)
    o = jnp.dot(t, w_ref[...], preferred_element_type=jnp.float32) + bl_ref[...]
    o_ref[...] = jnp.maximum(o, 0.0)


_k3 = pl.pallas_call(
    _k3_body,
    grid=(NP // BM,),
    in_specs=[
        pl.BlockSpec((BM, D), lambda i: (i, 0)),
        pl.BlockSpec((NC, BM, D), lambda i: (0, i, 0)),
        pl.BlockSpec((BM, 1), lambda i: (i, 0)),
        pl.BlockSpec((1, D), lambda i: (0, 0)),
        pl.BlockSpec((D, D), lambda i: (0, 0)),
        pl.BlockSpec((1, D), lambda i: (0, 0)),
    ],
    out_specs=pl.BlockSpec((BM, D), lambda i: (i, 0)),
    out_shape=jax.ShapeDtypeStruct((NP, D), jnp.float32),
)


def kernel(x, edge_index, W1, b1, W2, b2, Wl, bl):
    edge_index = edge_index.astype(jnp.int32)
    x_pad = jnp.pad(x, ((0, NP - N_NODES), (0, 0)))
    pad = jnp.full((EP - E,), N_NODES, jnp.int32)
    srcr = jnp.concatenate([edge_index[0], pad]).reshape(TOT_CH, K)
    dstr = jnp.concatenate([edge_index[1], pad]).reshape(TOT_CH, K)

    degp = _deg_call(dstr)
    hs1, dinv = _k1(x_pad, W1, degp)
    p = _edge_call(hs1, srcr, dstr)
    hs2 = _k2(hs1, p, dinv, b1.reshape(1, D), W2)
    q = _edge_call(hs2, srcr, dstr)
    out = _k3(hs2, q, dinv, b2.reshape(1, D), Wl, bl.reshape(1, D))
    return out[:N_NODES]


# speedup vs baseline: 1.4367x; 1.4367x over previous
import jax
import jax.numpy as jnp
from jax import lax
from jax.experimental import pallas as pl
from jax.experimental.pallas import tpu as pltpu
from jax.experimental.pallas import tpu_sc as plsc

N_NODES = 10000
D = 128
E = 320000

NC = 2
NS = 16
NW = NC * NS
K = 128
CH = 80
EW = K * CH
EP = EW * NW
NP = 10240
RPT = NP // NS
BM = 1024
TOT_CH = EP // K
GRP = 8
N_C0 = 152
N_C1 = 8

_SC_MESH = plsc.VectorSubcoreMesh(
    core_axis_name="c", subcore_axis_name="s", num_cores=NC, num_subcores=NS
)


def _deg_body(dstr, out, degw, dstv, onesb):
    c = lax.axis_index("c")
    s = lax.axis_index("s")
    wid = c * NS + s
    pltpu.sync_copy(dstr.at[pl.ds(wid * CH, CH)], dstv)

    def _fill0(i, carry):
        for kk in range(D // 16):
            onesb[i, pl.ds(kk * 16, 16)] = jnp.zeros((16,), jnp.float32)
        return carry

    lax.fori_loop(0, K, _fill0, 0)
    base = s * RPT
    for z in range(RPT // K):
        pltpu.sync_copy(onesb, degw.at[pl.ds(base + z * K, K)])

    def _fill1(i, carry):
        for kk in range(D // 16):
            onesb[i, pl.ds(kk * 16, 16)] = jnp.ones((16,), jnp.float32)
        return carry

    lax.fori_loop(0, K, _fill1, 0)
    plsc.subcore_barrier()

    def _chunk(t, carry):
        pltpu.sync_copy(onesb, degw.at[dstv.at[t]], add=True)
        return carry

    lax.fori_loop(0, CH, _chunk, 0)
    plsc.subcore_barrier()
    for z in range(RPT // K):
        pltpu.sync_copy(degw.at[pl.ds(base + z * K, K)], onesb)
        pltpu.sync_copy(onesb, out.at[c, pl.ds(base + z * K, K)])


_deg_call = pl.kernel(
    _deg_body,
    out_type=jax.ShapeDtypeStruct((NC, NP, D), jnp.float32),
    mesh=_SC_MESH,
    scratch_types=[
        pltpu.VMEM_SHARED((NP, D), jnp.float32),
        pltpu.VMEM((CH, K), jnp.int32),
        pltpu.VMEM((K, D), jnp.float32),
    ],
)


def _edge_body(hs, srcr, dstr, out, acc, srcv, dstv, buf0, buf1, sem0, sem1, sem2, sem3):
    c = lax.axis_index("c")
    s = lax.axis_index("s")

    def _zrow(i, carry):
        for kk in range(D // 16):
            buf0[i, pl.ds(kk * 16, 16)] = jnp.zeros((16,), jnp.float32)
        return carry

    lax.fori_loop(0, K, _zrow, 0)
    base = s * RPT
    for z in range(RPT // K):
        pltpu.sync_copy(buf0, acc.at[pl.ds(base + z * K, K)])
    plsc.subcore_barrier()

    def _chunk(t, carry):
        j0 = 2 * t
        j1 = 2 * t + 1
        pltpu.make_async_copy(hs.at[srcv.at[j0]], buf0, sem0).wait()
        pltpu.async_copy(buf0, acc.at[dstv.at[j0]], sem2, add=True)
        pltpu.make_async_copy(hs.at[srcv.at[j1]], buf1, sem1).wait()
        pltpu.async_copy(buf1, acc.at[dstv.at[j1]], sem3, add=True)
        pltpu.make_async_copy(buf0, acc.at[dstv.at[j0]], sem2).wait()
        pltpu.async_copy(hs.at[srcv.at[j0 + 2]], buf0, sem0)
        pltpu.make_async_copy(buf1, acc.at[dstv.at[j1]], sem3).wait()
        pltpu.async_copy(hs.at[srcv.at[j1 + 2]], buf1, sem1)
        return carry

    def _run(base, n_chunks):
        for g in range(n_chunks // GRP):
            gb = base + g * GRP
            pltpu.sync_copy(srcr.at[pl.ds(gb, GRP)], srcv)
            pltpu.sync_copy(dstr.at[pl.ds(gb, GRP)], dstv)
            pltpu.async_copy(hs.at[srcv.at[0]], buf0, sem0)
            pltpu.async_copy(hs.at[srcv.at[1]], buf1, sem1)
            lax.fori_loop(0, GRP // 2 - 1, _chunk, 0)
            pltpu.make_async_copy(hs.at[srcv.at[GRP - 2]], buf0, sem0).wait()
            pltpu.sync_copy(buf0, acc.at[dstv.at[GRP - 2]], add=True)
            pltpu.make_async_copy(hs.at[srcv.at[GRP - 1]], buf1, sem1).wait()
            pltpu.sync_copy(buf1, acc.at[dstv.at[GRP - 1]], add=True)

    @pl.when(c == 0)
    def _core0():
        _run(s * N_C0, N_C0)

    @pl.when(c == 1)
    def _core1():
        _run(NS * N_C0 + s * N_C1, N_C1)

    plsc.subcore_barrier()
    for z in range(RPT // K):
        pltpu.sync_copy(acc.at[pl.ds(base + z * K, K)], buf0)
        pltpu.sync_copy(buf0, out.at[c, pl.ds(base + z * K, K)])


_edge_call = pl.kernel(
    _edge_body,
    out_type=jax.ShapeDtypeStruct((NC, NP, D), jnp.float32),
    mesh=_SC_MESH,
    scratch_types=[
        pltpu.VMEM_SHARED((NP, D), jnp.float32),
        pltpu.VMEM((GRP, K), jnp.int32),
        pltpu.VMEM((GRP, K), jnp.int32),
        pltpu.VMEM((K, D), jnp.float32),
        pltpu.VMEM((K, D), jnp.float32),
        pltpu.SemaphoreType.DMA,
        pltpu.SemaphoreType.DMA,
        pltpu.SemaphoreType.DMA,
        pltpu.SemaphoreType.DMA,
    ],
)


def _k1_body(x_ref, w_ref, deg_ref, hs_ref, dinv_ref):
    d = deg_ref[0, :, 0:1] + deg_ref[1, :, 0:1] + 1.0
    dv = lax.rsqrt(d)
    dinv_ref[...] = dv
    h = jnp.dot(x_ref[...], w_ref[...], preferred_element_type=jnp.float32)
    hs_ref[...] = h * dv


_k1 = pl.pallas_call(
    _k1_body,
    grid=(NP // BM,),
    in_specs=[
        pl.BlockSpec((BM, D), lambda i: (i, 0)),
        pl.BlockSpec((D, D), lambda i: (0, 0)),
        pl.BlockSpec((NC, BM, D), lambda i: (0, i, 0)),
    ],
    out_specs=[
        pl.BlockSpec((BM, D), lambda i: (i, 0)),
        pl.BlockSpec((BM, 1), lambda i: (i, 0)),
    ],
    out_shape=[
        jax.ShapeDtypeStruct((NP, D), jnp.float32),
        jax.ShapeDtypeStruct((NP, 1), jnp.float32),
    ],
)


def _k2_body(hs_ref, p_ref, dinv_ref, b_ref, w_ref, o_ref):
    dv = dinv_ref[...]
    a = (hs_ref[...] + p_ref[0] + p_ref[1]) * dv + b_ref[...]
    t = jnp.maximum(a, 0.0)
    o_ref[...] = jnp.dot(t, w_ref[...], preferred_element_type=jnp.float32) * dv


_k2 = pl.pallas_call(
    _k2_body,
    grid=(NP // BM,),
    in_specs=[
        pl.BlockSpec((BM, D), lambda i: (i, 0)),
        pl.BlockSpec((NC, BM, D), lambda i: (0, i, 0)),
        pl.BlockSpec((BM, 1), lambda i: (i, 0)),
        pl.BlockSpec((1, D), lambda i: (0, 0)),
        pl.BlockSpec((D, D), lambda i: (0, 0)),
    ],
    out_specs=pl.BlockSpec((BM, D), lambda i: (i, 0)),
    out_shape=jax.ShapeDtypeStruct((NP, D), jnp.float32),
)


def _k3_body(hs_ref, p_ref, dinv_ref, b_ref, w_ref, bl_ref, o_ref):
    a = (hs_ref[...] + p_ref[0] + p_ref[1]) * dinv_ref[...] + b_ref[...]
    t = jnp.maximum(a, 0.0)
    o = jnp.dot(t, w_ref[...], preferred_element_type=jnp.float32) + bl_ref[...]
    o_ref[...] = jnp.maximum(o, 0.0)


_k3 = pl.pallas_call(
    _k3_body,
    grid=(NP // BM,),
    in_specs=[
        pl.BlockSpec((BM, D), lambda i: (i, 0)),
        pl.BlockSpec((NC, BM, D), lambda i: (0, i, 0)),
        pl.BlockSpec((BM, 1), lambda i: (i, 0)),
        pl.BlockSpec((1, D), lambda i: (0, 0)),
        pl.BlockSpec((D, D), lambda i: (0, 0)),
        pl.BlockSpec((1, D), lambda i: (0, 0)),
    ],
    out_specs=pl.BlockSpec((BM, D), lambda i: (i, 0)),
    out_shape=jax.ShapeDtypeStruct((NP, D), jnp.float32),
)


def kernel(x, edge_index, W1, b1, W2, b2, Wl, bl):
    edge_index = edge_index.astype(jnp.int32)
    x_pad = jnp.pad(x, ((0, NP - N_NODES), (0, 0)))
    pad = jnp.full((EP - E,), N_NODES, jnp.int32)
    srcr = jnp.concatenate([edge_index[0], pad]).reshape(TOT_CH, K)
    dstr = jnp.concatenate([edge_index[1], pad]).reshape(TOT_CH, K)

    degp = _deg_call(dstr)
    hs1, dinv = _k1(x_pad, W1, degp)
    p = _edge_call(hs1, srcr, dstr)
    hs2 = _k2(hs1, p, dinv, b1.reshape(1, D), W2)
    q = _edge_call(hs2, srcr, dstr)
    out = _k3(hs2, q, dinv, b2.reshape(1, D), Wl, bl.reshape(1, D))
    return out[:N_NODES]
